# Initial kernel scaffold; baseline (speedup 1.0000x reference)
import jax
import jax.numpy as jnp
from jax.experimental import pallas as pl


def kernel(input, item_table, cat_table, shop_table, bn0_g, bn0_b, fc1_w, fc1_b, bn1_g, bn1_b, fc2_w, fc2_b, bn2_g, bn2_b, out_w, out_b):
    raise NotImplementedError("write your pallas kernel here")


# speedup vs baseline: 1.1791x; 1.1791x over previous
import functools

import jax
import jax.numpy as jnp
from jax import lax
from jax.experimental import pallas as pl
from jax.experimental.pallas import tpu as pltpu
from jax.experimental.pallas import tpu_sc as plsc

B = 16384
D_ITEM = 64
D_PAD = 16
EPS = 1e-5

_NC = 2
_NS = 16
_NW = _NC * _NS
_BPW = B // _NW
_CH = 128
_NCH = _BPW // _CH
_ROWS = _NW * _NCH


def _sc_gather(idx1, idx2, idx3, item_t, cat_t, shop_t):
    mesh = plsc.VectorSubcoreMesh(core_axis_name="c", subcore_axis_name="s")

    @functools.partial(
        pl.kernel,
        mesh=mesh,
        compiler_params=pltpu.CompilerParams(use_tc_tiling_on_sc=False),
        out_type=[
            jax.ShapeDtypeStruct((_ROWS, _CH, D_ITEM), jnp.float32),
            jax.ShapeDtypeStruct((_ROWS, _CH, D_PAD), jnp.float32),
            jax.ShapeDtypeStruct((_ROWS, _CH, D_PAD), jnp.float32),
        ],
        scratch_types=[
            pltpu.VMEM((_NCH, _CH), jnp.int32),
            pltpu.VMEM((_NCH, _CH), jnp.int32),
            pltpu.VMEM((_NCH, _CH), jnp.int32),
            pltpu.VMEM((_NCH, _CH, D_ITEM), jnp.float32),
            pltpu.VMEM((_NCH, _CH, D_PAD), jnp.float32),
            pltpu.VMEM((_NCH, _CH, D_PAD), jnp.float32),
            pltpu.SemaphoreType.DMA,
            pltpu.SemaphoreType.DMA,
            pltpu.SemaphoreType.DMA,
        ],
    )
    def k(idx1_h, idx2_h, idx3_h, item_h, cat_h, shop_h,
          t1_h, t2_h, t3_h,
          i1_v, i2_v, i3_v, r1_v, r2_v, r3_v, s1, s2, s3):
        wid = lax.axis_index("s") * _NC + lax.axis_index("c")
        row0 = wid * _NCH
        pltpu.sync_copy(idx1_h.at[pl.ds(row0, _NCH)], i1_v)
        pltpu.sync_copy(idx2_h.at[pl.ds(row0, _NCH)], i2_v)
        pltpu.sync_copy(idx3_h.at[pl.ds(row0, _NCH)], i3_v)
        copies = []
        for j in range(_NCH):
            copies.append(pltpu.async_copy(item_h.at[i1_v.at[j]], r1_v.at[j], s1))
            copies.append(pltpu.async_copy(cat_h.at[i2_v.at[j]], r2_v.at[j], s2))
            copies.append(pltpu.async_copy(shop_h.at[i3_v.at[j]], r3_v.at[j], s3))
        for c in copies:
            c.wait()
        pltpu.sync_copy(r1_v, t1_h.at[pl.ds(row0, _NCH)])
        pltpu.sync_copy(r2_v, t2_h.at[pl.ds(row0, _NCH)])
        pltpu.sync_copy(r3_v, t3_h.at[pl.ds(row0, _NCH)])

    return k(idx1, idx2, idx3, item_t, cat_t, shop_t)


def _mlp_body(t1_r, t2_r, t3_r, w1a_r, w1b_r, w1c_r, p_r, w2_r, w3_r, o_r):
    def bn(x, g, b):
        m = jnp.mean(x, axis=0, keepdims=True)
        v = jnp.mean((x - m) ** 2, axis=0, keepdims=True)
        return (x - m) * lax.rsqrt(v + EPS) * g + b

    p = p_r[...]
    x1 = bn(t1_r[...], p[0:1, 0:D_ITEM], p[1:2, 0:D_ITEM])
    x2 = bn(t2_r[...], p[2:3, 0:D_PAD], p[3:4, 0:D_PAD])
    x3 = bn(t3_r[...], p[4:5, 0:D_PAD], p[5:6, 0:D_PAD])
    h = (jnp.dot(x1, w1a_r[...], preferred_element_type=jnp.float32)
         + jnp.dot(x2, w1b_r[...], preferred_element_type=jnp.float32)
         + jnp.dot(x3, w1c_r[...], preferred_element_type=jnp.float32))
    h = jax.nn.relu(h + p[6:7, 0:40])
    h = bn(h, p[7:8, 0:40], p[8:9, 0:40])
    h = jnp.dot(h, w2_r[...], preferred_element_type=jnp.float32)
    h = jax.nn.relu(h + p[9:10, 0:20])
    h = bn(h, p[10:11, 0:20], p[11:12, 0:20])
    y = jnp.dot(h, w3_r[...], preferred_element_type=jnp.float32)
    o_r[...] = y + p[12:13, 0:1]


def kernel(input, item_table, cat_table, shop_table, bn0_g, bn0_b,
           fc1_w, fc1_b, bn1_g, bn1_b, fc2_w, fc2_b, bn2_g, bn2_b,
           out_w, out_b):
    idx = input.astype(jnp.int32)
    idx1 = idx[:, 0].reshape(_ROWS, _CH)
    idx2 = idx[:, 1].reshape(_ROWS, _CH)
    idx3 = idx[:, 2].reshape(_ROWS, _CH)
    cat_p = jnp.pad(cat_table, ((0, 0), (0, D_PAD - cat_table.shape[1])))
    shop_p = jnp.pad(shop_table, ((0, 0), (0, D_PAD - shop_table.shape[1])))

    t1, t2, t3 = _sc_gather(idx1, idx2, idx3, item_table, cat_p, shop_p)
    t1 = t1.reshape(B, D_ITEM)
    t2 = t2.reshape(B, D_PAD)
    t3 = t3.reshape(B, D_PAD)

    w1a = fc1_w[:, 0:D_ITEM].T
    w1b = jnp.pad(fc1_w[:, 64:72].T, ((0, 8), (0, 0)))
    w1c = jnp.pad(fc1_w[:, 72:80].T, ((0, 8), (0, 0)))
    w2 = fc2_w.T
    w3 = out_w.T

    def row(vec):
        return jnp.pad(vec, (0, 64 - vec.shape[0]))
    params = jnp.stack([
        row(bn0_g[0:D_ITEM]), row(bn0_b[0:D_ITEM]),
        row(jnp.pad(bn0_g[64:72], (0, 8), constant_values=1.0)),
        row(jnp.pad(bn0_b[64:72], (0, 8))),
        row(jnp.pad(bn0_g[72:80], (0, 8), constant_values=1.0)),
        row(jnp.pad(bn0_b[72:80], (0, 8))),
        row(fc1_b), row(bn1_g), row(bn1_b),
        row(fc2_b), row(bn2_g), row(bn2_b),
        row(out_b),
    ])

    y = pl.pallas_call(
        _mlp_body,
        out_shape=jax.ShapeDtypeStruct((B, 1), jnp.float32),
    )(t1, t2, t3, w1a, w1b, w1c, params, w2, w3)
    return y[:, 0]
